# baseline (speedup 1.0000x reference)
import jax
import jax.numpy as jnp
from jax.experimental import pallas as pl
from jax.experimental.pallas import tpu as pltpu

N = 10000
D_IN = 128
D_OUT = 128
BLOCK = 200
NSLABS = N // BLOCK
NBUF = 4


def _gnn_kernel(feat_ref, adj_ref, w_ref, out_ref, bufs_ref, support_ref, sems):
    def copy(slab, slot):
        return pltpu.make_async_copy(
            adj_ref.at[pl.ds(slab * BLOCK, BLOCK), :],
            bufs_ref.at[slot],
            sems.at[slot],
        )

    for s in range(NBUF):
        copy(s, s).start()

    support_ref[...] = jnp.dot(
        feat_ref[...], w_ref[...], preferred_element_type=jnp.float32
    )

    def body(s, carry):
        slot = jax.lax.rem(s, NBUF)
        copy(s, slot).wait()
        acc = jnp.dot(
            bufs_ref[slot], support_ref[...], preferred_element_type=jnp.float32
        )
        out_ref[pl.ds(s * BLOCK, BLOCK), :] = jnp.maximum(acc, 0.0)

        @pl.when(s + NBUF < NSLABS)
        def _():
            copy(s + NBUF, slot).start()

        return carry

    jax.lax.fori_loop(0, NSLABS, body, 0)


def kernel(features, adj, W):
    return pl.pallas_call(
        _gnn_kernel,
        in_specs=[
            pl.BlockSpec(memory_space=pltpu.VMEM),
            pl.BlockSpec(memory_space=pltpu.MemorySpace.HBM),
            pl.BlockSpec(memory_space=pltpu.VMEM),
        ],
        out_specs=pl.BlockSpec(memory_space=pltpu.VMEM),
        out_shape=jax.ShapeDtypeStruct((N, D_OUT), jnp.float32),
        scratch_shapes=[
            pltpu.VMEM((NBUF, BLOCK, N), jnp.float32),
            pltpu.VMEM((N, D_OUT), jnp.float32),
            pltpu.SemaphoreType.DMA((NBUF,)),
        ],
    )(features, adj, W)
